# baseline (speedup 1.0000x reference)
import functools

import jax
import jax.numpy as jnp
from jax import lax
from jax.experimental import pallas as pl
from jax.experimental.pallas import tpu as pltpu
from jax.experimental.pallas import tpu_sc as plsc

N = 10000
IN_CH = 128
HID = 16
HEADS = 8
OUT_CH = 128

NC = 2
NS = 16
NW = NC * NS
EB = 96
ICH = 18
NBLK0 = 90
NBLK1 = 126
TOTBLK = NS * (NBLK0 + NBLK1)
EPAD = TOTBLK * EB
NPAD = 10240
ROW = 144
RPT = NPAD // NS

_mesh = plsc.VectorSubcoreMesh(core_axis_name="c", subcore_axis_name="s",
                               num_cores=NC, num_subcores=NS)

_GATHER_DN = lax.GatherDimensionNumbers(
    offset_dims=(), collapsed_slice_dims=(0,), start_index_map=(0,))


def _bcast_lane(vec, lane):
  idx = jnp.full((16, 1), lane, jnp.int32)
  return lax.gather(vec, idx, _GATHER_DN, (1,),
                    mode=lax.GatherScatterMode.PROMISE_IN_BOUNDS)


def _edge_body(nheads, tabA, tabB, srcb, dstb, out, idx_s, idx_d, g0, g1,
               d0, d1, acc, sem0, sem1, ssem0, ssem1):
  c = lax.axis_index("c")
  s = lax.axis_index("s")
  start_blk = jnp.where(c == 0, s * NBLK0, NS * NBLK0 + s * NBLK1)
  n_chunks = jnp.where(c == 0, NBLK0 // ICH, NBLK1 // ICH)
  gs = (g0, g1)
  dds = (d0, d1)
  sems = (sem0, sem1)
  ssems = (ssem0, ssem1)

  def zero_row(r, carry):
    for j in range(ROW // 16):
      g0[r, pl.ds(j * 16, 16)] = jnp.zeros((16,), jnp.float32)
    return carry
  lax.fori_loop(0, EB, zero_row, 0)
  for k in range(RPT // EB):
    pltpu.sync_copy(g0, acc.at[pl.ds(s * RPT + k * EB, EB)])
  rem = RPT - (RPT // EB) * EB
  if rem:
    pltpu.sync_copy(g0.at[pl.ds(0, rem)],
                    acc.at[pl.ds(s * RPT + (RPT // EB) * EB, rem)])

  plsc.subcore_barrier()

  lanes = lax.iota(jnp.int32, 16)
  keep = lanes < nheads

  def do_chunk(ch, carry):
    pltpu.sync_copy(srcb.at[pl.ds(start_blk + ch * ICH, ICH)], idx_s)
    pltpu.sync_copy(dstb.at[pl.ds(start_blk + ch * ICH, ICH)], idx_d)
    pltpu.async_copy(tabA.at[idx_s.at[0]], g0, sem0)
    pltpu.async_copy(tabB.at[idx_d.at[0]], d0, sem0)

    def do_pair(t, pcarry):
      for q in (0, 1):
        b = t * 2 + q
        nq = 1 - q

        @pl.when(b + 1 < ICH)
        def _prefetch(b=b, nq=nq):
          @pl.when(b >= 1)
          def _drain(nq=nq):
            pltpu.make_async_copy(gs[nq], acc.at[pl.ds(0, EB)],
                                  ssems[nq]).wait()
          pltpu.async_copy(tabA.at[idx_s.at[b + 1]], gs[nq], sems[nq])
          pltpu.async_copy(tabB.at[idx_d.at[b + 1]], dds[nq], sems[nq])

        pltpu.make_async_copy(tabA.at[pl.ds(0, EB)], gs[q], sems[q]).wait()
        pltpu.make_async_copy(tabB.at[pl.ds(0, EB)], dds[q], sems[q]).wait()

        g = gs[q]
        dd = dds[q]

        @plsc.parallel_loop(0, EB, step=1, unroll=4)
        def do_edge(e):
          u = g[e, pl.ds(128, 16)]
          v = dd[e, :]
          w = u + v
          a = jnp.where(w > 0, w, 0.2 * w)
          p = jnp.where(keep, jnp.exp(a), 0.0)
          for j in range(8):
            ph = _bcast_lane(p, j if nheads == HEADS else 0)
            g[e, pl.ds(j * 16, 16)] = ph * g[e, pl.ds(j * 16, 16)]
          g[e, pl.ds(128, 16)] = p

        pltpu.async_copy(g, acc.at[idx_d.at[b]], ssems[q], add=True)
      return pcarry
    lax.fori_loop(0, ICH // 2, do_pair, 0)
    pltpu.make_async_copy(g0, acc.at[pl.ds(0, EB)], ssem0).wait()
    pltpu.make_async_copy(g1, acc.at[pl.ds(0, EB)], ssem1).wait()
    return carry
  lax.fori_loop(0, n_chunks, do_chunk, 0)

  plsc.subcore_barrier()
  pltpu.sync_copy(acc.at[pl.ds(s * RPT, RPT)],
                  out.at[c, pl.ds(s * RPT, RPT)])


def _make_edge_pass(nheads):
  return functools.partial(
      pl.kernel,
      out_type=jax.ShapeDtypeStruct((NC, NPAD, ROW), jnp.float32),
      mesh=_mesh,
      scratch_types=[
          pltpu.VMEM((ICH, EB), jnp.int32),
          pltpu.VMEM((ICH, EB), jnp.int32),
          pltpu.VMEM((EB, ROW), jnp.float32),
          pltpu.VMEM((EB, ROW), jnp.float32),
          pltpu.VMEM((EB, 16), jnp.float32),
          pltpu.VMEM((EB, 16), jnp.float32),
          pltpu.VMEM_SHARED((NPAD, ROW), jnp.float32),
          pltpu.SemaphoreType.DMA,
          pltpu.SemaphoreType.DMA,
          pltpu.SemaphoreType.DMA,
          pltpu.SemaphoreType.DMA,
      ],
      compiler_params=pltpu.CompilerParams(use_tc_tiling_on_sc=False),
  )(functools.partial(_edge_body, nheads))


_edge_pass8 = _make_edge_pass(HEADS)
_edge_pass1 = _make_edge_pass(1)

_BN = 512


def _mm_body(x_ref, wa_ref, wb_ref, oa_ref, ob_ref):
  x = x_ref[...]
  oa_ref[...] = jnp.dot(x, wa_ref[...], preferred_element_type=jnp.float32,
                        precision=lax.Precision.HIGHEST)
  ob_ref[...] = jnp.dot(x, wb_ref[...], preferred_element_type=jnp.float32,
                        precision=lax.Precision.HIGHEST)


def _tables1(xp, wa, wb):
  return pl.pallas_call(
      _mm_body,
      grid=(NPAD // _BN,),
      in_specs=[
          pl.BlockSpec((_BN, IN_CH), lambda i: (i, 0)),
          pl.BlockSpec((IN_CH, ROW), lambda i: (0, 0)),
          pl.BlockSpec((IN_CH, 16), lambda i: (0, 0)),
      ],
      out_specs=[
          pl.BlockSpec((_BN, ROW), lambda i: (i, 0)),
          pl.BlockSpec((_BN, 16), lambda i: (i, 0)),
      ],
      out_shape=[
          jax.ShapeDtypeStruct((NPAD, ROW), jnp.float32),
          jax.ShapeDtypeStruct((NPAD, 16), jnp.float32),
      ],
  )(xp, wa, wb)


def _prep2_body(a0_ref, a1_ref, r_ref, b1_ref, wa_ref, wb_ref,
                oa_ref, ob_ref):
  acc = a0_ref[...] + a1_ref[...]
  asum = acc[:, 128:136]
  d = jnp.dot(asum, r_ref[...], preferred_element_type=jnp.float32,
              precision=lax.Precision.HIGHEST) + 1e-16
  h1 = acc[:, :128] / d + b1_ref[...]
  e1 = jnp.where(h1 > 0, h1, jnp.exp(h1) - 1.0)
  oa_ref[...] = jnp.dot(e1, wa_ref[...], preferred_element_type=jnp.float32,
                        precision=lax.Precision.HIGHEST)
  ob_ref[...] = jnp.dot(e1, wb_ref[...], preferred_element_type=jnp.float32,
                        precision=lax.Precision.HIGHEST)


def _tables2(a0, a1, r, b1, wa, wb):
  return pl.pallas_call(
      _prep2_body,
      grid=(NPAD // _BN,),
      in_specs=[
          pl.BlockSpec((_BN, ROW), lambda i: (i, 0)),
          pl.BlockSpec((_BN, ROW), lambda i: (i, 0)),
          pl.BlockSpec((HEADS, 128), lambda i: (0, 0)),
          pl.BlockSpec((1, 128), lambda i: (0, 0)),
          pl.BlockSpec((IN_CH, ROW), lambda i: (0, 0)),
          pl.BlockSpec((IN_CH, 16), lambda i: (0, 0)),
      ],
      out_specs=[
          pl.BlockSpec((_BN, ROW), lambda i: (i, 0)),
          pl.BlockSpec((_BN, 16), lambda i: (i, 0)),
      ],
      out_shape=[
          jax.ShapeDtypeStruct((NPAD, ROW), jnp.float32),
          jax.ShapeDtypeStruct((NPAD, 16), jnp.float32),
      ],
  )(a0, a1, r, b1, wa, wb)


def _final_body(a0_ref, a1_ref, b2_ref, o_ref):
  acc = a0_ref[...] + a1_ref[...]
  d = acc[:, 128:129] + 1e-16
  o_ref[...] = acc[:, :128] / d + b2_ref[...]


def _final(a0, a1, b2):
  return pl.pallas_call(
      _final_body,
      grid=(NPAD // _BN,),
      in_specs=[
          pl.BlockSpec((_BN, ROW), lambda i: (i, 0)),
          pl.BlockSpec((_BN, ROW), lambda i: (i, 0)),
          pl.BlockSpec((1, 128), lambda i: (0, 0)),
      ],
      out_specs=pl.BlockSpec((_BN, 128), lambda i: (i, 0)),
      out_shape=jax.ShapeDtypeStruct((NPAD, 128), jnp.float32),
  )(a0, a1, b2)


@jax.jit
def kernel(x, W1, att_src1, att_dst1, b1, W2, att_src2, att_dst2, b2,
           edge_index):
  f32 = jnp.float32
  as1 = att_src1.reshape(HEADS, HID)
  ad1 = att_dst1.reshape(HEADS, HID)
  eye8 = jnp.eye(HEADS, dtype=f32)
  A_s = (eye8[:, None, :] * as1[:, :, None]).reshape(IN_CH, HEADS)
  A_d = (eye8[:, None, :] * ad1[:, :, None]).reshape(IN_CH, HEADS)
  w1a = jnp.concatenate([W1, W1 @ A_s, W1 @ A_d], axis=1)
  w1b = jnp.concatenate([W1 @ A_d, jnp.zeros((IN_CH, 8), f32)], axis=1)
  a_s2 = att_src2.reshape(OUT_CH)
  a_d2 = att_dst2.reshape(OUT_CH)
  w2a = jnp.concatenate(
      [W2, (W2 @ a_s2)[:, None], jnp.zeros((HEADS * HID, 15), f32)], axis=1)
  w2b = jnp.concatenate(
      [(W2 @ a_d2)[:, None], jnp.zeros((HEADS * HID, 15), f32)], axis=1)
  r8 = jnp.repeat(eye8, HID, axis=1)

  loop = jnp.arange(N, dtype=jnp.int32)
  tail = jnp.full((EPAD - N - edge_index.shape[1],), N, jnp.int32)
  src = jnp.concatenate([edge_index[0], loop, tail]).reshape(TOTBLK, EB)
  dst = jnp.concatenate([edge_index[1], loop, tail]).reshape(TOTBLK, EB)

  xp = jnp.pad(x, ((0, NPAD - N), (0, 0)))

  tabA1, tabB1 = _tables1(xp, w1a, w1b)
  acc1 = _edge_pass8(tabA1, tabB1, src, dst)
  tabA2, tabB2 = _tables2(acc1[0], acc1[1], r8, b1.reshape(1, 128), w2a, w2b)
  acc2 = _edge_pass1(tabA2, tabB2, src, dst)
  out = _final(acc2[0], acc2[1], b2.reshape(1, 128))
  return out[:N]


# speedup vs baseline: 1.1208x; 1.1208x over previous
import functools

import jax
import jax.numpy as jnp
from jax import lax
from jax.experimental import pallas as pl
from jax.experimental.pallas import tpu as pltpu
from jax.experimental.pallas import tpu_sc as plsc

N = 10000
IN_CH = 128
HID = 16
HEADS = 8
OUT_CH = 128

NC = 2
NS = 16
NW = NC * NS
EB = 96
ICH = 18
NBLK0 = 126
NBLK1 = 90
TOTBLK = NS * (NBLK0 + NBLK1)
EPAD = TOTBLK * EB
NPAD = 10240
ROW = 144
RPT = NPAD // NS

_mesh = plsc.VectorSubcoreMesh(core_axis_name="c", subcore_axis_name="s",
                               num_cores=NC, num_subcores=NS)

_GATHER_DN = lax.GatherDimensionNumbers(
    offset_dims=(), collapsed_slice_dims=(0,), start_index_map=(0,))


def _bcast_lane(vec, lane):
  idx = jnp.full((16, 1), lane, jnp.int32)
  return lax.gather(vec, idx, _GATHER_DN, (1,),
                    mode=lax.GatherScatterMode.PROMISE_IN_BOUNDS)


def _edge_body(nheads, tabA, tabB, srcb, dstb, out, idx_s, idx_d, g0, g1,
               d0, d1, acc, sem0, sem1, ssem0, ssem1):
  c = lax.axis_index("c")
  s = lax.axis_index("s")
  start_blk = jnp.where(c == 0, s * NBLK0, NS * NBLK0 + s * NBLK1)
  n_chunks = jnp.where(c == 0, NBLK0 // ICH, NBLK1 // ICH)
  gs = (g0, g1)
  dds = (d0, d1)
  sems = (sem0, sem1)
  ssems = (ssem0, ssem1)

  def zero_row(r, carry):
    for j in range(ROW // 16):
      g0[r, pl.ds(j * 16, 16)] = jnp.zeros((16,), jnp.float32)
    return carry
  lax.fori_loop(0, EB, zero_row, 0)
  for k in range(RPT // EB):
    pltpu.sync_copy(g0, acc.at[pl.ds(s * RPT + k * EB, EB)])
  rem = RPT - (RPT // EB) * EB
  if rem:
    pltpu.sync_copy(g0.at[pl.ds(0, rem)],
                    acc.at[pl.ds(s * RPT + (RPT // EB) * EB, rem)])

  plsc.subcore_barrier()

  lanes = lax.iota(jnp.int32, 16)
  keep = lanes < nheads

  def do_chunk(ch, carry):
    pltpu.sync_copy(srcb.at[pl.ds(start_blk + ch * ICH, ICH)], idx_s)
    pltpu.sync_copy(dstb.at[pl.ds(start_blk + ch * ICH, ICH)], idx_d)
    pltpu.async_copy(tabA.at[idx_s.at[0]], g0, sem0)
    pltpu.async_copy(tabB.at[idx_d.at[0]], d0, sem0)

    def do_pair(t, pcarry):
      for q in (0, 1):
        b = t * 2 + q
        nq = 1 - q

        @pl.when(b + 1 < ICH)
        def _prefetch(b=b, nq=nq):
          @pl.when(b >= 1)
          def _drain(nq=nq):
            pltpu.make_async_copy(gs[nq], acc.at[pl.ds(0, EB)],
                                  ssems[nq]).wait()
          pltpu.async_copy(tabA.at[idx_s.at[b + 1]], gs[nq], sems[nq])
          pltpu.async_copy(tabB.at[idx_d.at[b + 1]], dds[nq], sems[nq])

        pltpu.make_async_copy(tabA.at[pl.ds(0, EB)], gs[q], sems[q]).wait()
        pltpu.make_async_copy(tabB.at[pl.ds(0, EB)], dds[q], sems[q]).wait()

        g = gs[q]
        dd = dds[q]

        @plsc.parallel_loop(0, EB, step=1, unroll=4)
        def do_edge(e):
          u = g[e, pl.ds(128, 16)]
          v = dd[e, :]
          w = u + v
          a = jnp.where(w > 0, w, 0.2 * w)
          p = jnp.where(keep, jnp.exp(a), 0.0)
          for j in range(8):
            ph = _bcast_lane(p, j if nheads == HEADS else 0)
            g[e, pl.ds(j * 16, 16)] = ph * g[e, pl.ds(j * 16, 16)]
          g[e, pl.ds(128, 16)] = p

        pltpu.async_copy(g, acc.at[idx_d.at[b]], ssems[q], add=True)
      return pcarry
    lax.fori_loop(0, ICH // 2, do_pair, 0)
    pltpu.make_async_copy(g0, acc.at[pl.ds(0, EB)], ssem0).wait()
    pltpu.make_async_copy(g1, acc.at[pl.ds(0, EB)], ssem1).wait()
    return carry
  lax.fori_loop(0, n_chunks, do_chunk, 0)

  plsc.subcore_barrier()
  pltpu.sync_copy(acc.at[pl.ds(s * RPT, RPT)],
                  out.at[c, pl.ds(s * RPT, RPT)])


def _make_edge_pass(nheads):
  return functools.partial(
      pl.kernel,
      out_type=jax.ShapeDtypeStruct((NC, NPAD, ROW), jnp.float32),
      mesh=_mesh,
      scratch_types=[
          pltpu.VMEM((ICH, EB), jnp.int32),
          pltpu.VMEM((ICH, EB), jnp.int32),
          pltpu.VMEM((EB, ROW), jnp.float32),
          pltpu.VMEM((EB, ROW), jnp.float32),
          pltpu.VMEM((EB, 16), jnp.float32),
          pltpu.VMEM((EB, 16), jnp.float32),
          pltpu.VMEM_SHARED((NPAD, ROW), jnp.float32),
          pltpu.SemaphoreType.DMA,
          pltpu.SemaphoreType.DMA,
          pltpu.SemaphoreType.DMA,
          pltpu.SemaphoreType.DMA,
      ],
      compiler_params=pltpu.CompilerParams(use_tc_tiling_on_sc=False),
  )(functools.partial(_edge_body, nheads))


_edge_pass8 = _make_edge_pass(HEADS)
_edge_pass1 = _make_edge_pass(1)

_BN = 512


def _mm_body(x_ref, wa_ref, wb_ref, oa_ref, ob_ref):
  x = x_ref[...]
  oa_ref[...] = jnp.dot(x, wa_ref[...], preferred_element_type=jnp.float32,
                        precision=lax.Precision.HIGHEST)
  ob_ref[...] = jnp.dot(x, wb_ref[...], preferred_element_type=jnp.float32,
                        precision=lax.Precision.HIGHEST)


def _tables1(xp, wa, wb):
  return pl.pallas_call(
      _mm_body,
      grid=(NPAD // _BN,),
      in_specs=[
          pl.BlockSpec((_BN, IN_CH), lambda i: (i, 0)),
          pl.BlockSpec((IN_CH, ROW), lambda i: (0, 0)),
          pl.BlockSpec((IN_CH, 16), lambda i: (0, 0)),
      ],
      out_specs=[
          pl.BlockSpec((_BN, ROW), lambda i: (i, 0)),
          pl.BlockSpec((_BN, 16), lambda i: (i, 0)),
      ],
      out_shape=[
          jax.ShapeDtypeStruct((NPAD, ROW), jnp.float32),
          jax.ShapeDtypeStruct((NPAD, 16), jnp.float32),
      ],
  )(xp, wa, wb)


def _prep2_body(a0_ref, a1_ref, r_ref, b1_ref, wa_ref, wb_ref,
                oa_ref, ob_ref):
  acc = a0_ref[...] + a1_ref[...]
  asum = acc[:, 128:136]
  d = jnp.dot(asum, r_ref[...], preferred_element_type=jnp.float32,
              precision=lax.Precision.HIGHEST) + 1e-16
  h1 = acc[:, :128] / d + b1_ref[...]
  e1 = jnp.where(h1 > 0, h1, jnp.exp(h1) - 1.0)
  oa_ref[...] = jnp.dot(e1, wa_ref[...], preferred_element_type=jnp.float32,
                        precision=lax.Precision.HIGHEST)
  ob_ref[...] = jnp.dot(e1, wb_ref[...], preferred_element_type=jnp.float32,
                        precision=lax.Precision.HIGHEST)


def _tables2(a0, a1, r, b1, wa, wb):
  return pl.pallas_call(
      _prep2_body,
      grid=(NPAD // _BN,),
      in_specs=[
          pl.BlockSpec((_BN, ROW), lambda i: (i, 0)),
          pl.BlockSpec((_BN, ROW), lambda i: (i, 0)),
          pl.BlockSpec((HEADS, 128), lambda i: (0, 0)),
          pl.BlockSpec((1, 128), lambda i: (0, 0)),
          pl.BlockSpec((IN_CH, ROW), lambda i: (0, 0)),
          pl.BlockSpec((IN_CH, 16), lambda i: (0, 0)),
      ],
      out_specs=[
          pl.BlockSpec((_BN, ROW), lambda i: (i, 0)),
          pl.BlockSpec((_BN, 16), lambda i: (i, 0)),
      ],
      out_shape=[
          jax.ShapeDtypeStruct((NPAD, ROW), jnp.float32),
          jax.ShapeDtypeStruct((NPAD, 16), jnp.float32),
      ],
  )(a0, a1, r, b1, wa, wb)


def _final_body(a0_ref, a1_ref, b2_ref, o_ref):
  acc = a0_ref[...] + a1_ref[...]
  d = acc[:, 128:129] + 1e-16
  o_ref[...] = acc[:, :128] / d + b2_ref[...]


def _final(a0, a1, b2):
  return pl.pallas_call(
      _final_body,
      grid=(NPAD // _BN,),
      in_specs=[
          pl.BlockSpec((_BN, ROW), lambda i: (i, 0)),
          pl.BlockSpec((_BN, ROW), lambda i: (i, 0)),
          pl.BlockSpec((1, 128), lambda i: (0, 0)),
      ],
      out_specs=pl.BlockSpec((_BN, 128), lambda i: (i, 0)),
      out_shape=jax.ShapeDtypeStruct((NPAD, 128), jnp.float32),
  )(a0, a1, b2)


@jax.jit
def kernel(x, W1, att_src1, att_dst1, b1, W2, att_src2, att_dst2, b2,
           edge_index):
  f32 = jnp.float32
  as1 = att_src1.reshape(HEADS, HID)
  ad1 = att_dst1.reshape(HEADS, HID)
  eye8 = jnp.eye(HEADS, dtype=f32)
  A_s = (eye8[:, None, :] * as1[:, :, None]).reshape(IN_CH, HEADS)
  A_d = (eye8[:, None, :] * ad1[:, :, None]).reshape(IN_CH, HEADS)
  w1a = jnp.concatenate([W1, W1 @ A_s, W1 @ A_d], axis=1)
  w1b = jnp.concatenate([W1 @ A_d, jnp.zeros((IN_CH, 8), f32)], axis=1)
  a_s2 = att_src2.reshape(OUT_CH)
  a_d2 = att_dst2.reshape(OUT_CH)
  w2a = jnp.concatenate(
      [W2, (W2 @ a_s2)[:, None], jnp.zeros((HEADS * HID, 15), f32)], axis=1)
  w2b = jnp.concatenate(
      [(W2 @ a_d2)[:, None], jnp.zeros((HEADS * HID, 15), f32)], axis=1)
  r8 = jnp.repeat(eye8, HID, axis=1)

  loop = jnp.arange(N, dtype=jnp.int32)
  tail = jnp.full((EPAD - N - edge_index.shape[1],), N, jnp.int32)
  src = jnp.concatenate([edge_index[0], loop, tail]).reshape(TOTBLK, EB)
  dst = jnp.concatenate([edge_index[1], loop, tail]).reshape(TOTBLK, EB)

  xp = jnp.pad(x, ((0, NPAD - N), (0, 0)))

  tabA1, tabB1 = _tables1(xp, w1a, w1b)
  acc1 = _edge_pass8(tabA1, tabB1, src, dst)
  tabA2, tabB2 = _tables2(acc1[0], acc1[1], r8, b1.reshape(1, 128), w2a, w2b)
  acc2 = _edge_pass1(tabA2, tabB2, src, dst)
  out = _final(acc2[0], acc2[1], b2.reshape(1, 128))
  return out[:N]


# speedup vs baseline: 1.2551x; 1.1199x over previous
import functools

import jax
import jax.numpy as jnp
from jax import lax
from jax.experimental import pallas as pl
from jax.experimental.pallas import tpu as pltpu
from jax.experimental.pallas import tpu_sc as plsc

N = 10000
IN_CH = 128
HID = 16
HEADS = 8
OUT_CH = 128

NC = 2
NS = 16
NW = NC * NS
EB = 96
ICH = 18
NBLK0 = 126
NBLK1 = 90
TOTBLK = NS * (NBLK0 + NBLK1)
EPAD = TOTBLK * EB
NPAD = 10240
ROW = 144
RPT = NPAD // NS

_HPERM = tuple(
    (2 * jg + (m % 2)) * 16 + (m // 2)
    for jg in range(4) for m in range(32))

_mesh = plsc.VectorSubcoreMesh(core_axis_name="c", subcore_axis_name="s",
                               num_cores=NC, num_subcores=NS)

_GATHER_DN = lax.GatherDimensionNumbers(
    offset_dims=(), collapsed_slice_dims=(0,), start_index_map=(0,))


def _bcast_lane(vec, lane):
  idx = jnp.full((16, 1), lane, jnp.int32)
  return lax.gather(vec, idx, _GATHER_DN, (1,),
                    mode=lax.GatherScatterMode.PROMISE_IN_BOUNDS)


def _edge_body(nheads, tabH, tabAt, tabB, srcb, dstb, out, idx_s, idx_d,
               gh0, gh1, ga0, ga1, gb0, gb1, obuf, acc, sem0, sem1):
  c = lax.axis_index("c")
  s = lax.axis_index("s")
  start_blk = jnp.where(c == 0, s * NBLK0, NS * NBLK0 + s * NBLK1)
  n_chunks = jnp.where(c == 0, NBLK0 // ICH, NBLK1 // ICH)
  ghs = (gh0, gh1)
  gas = (ga0, ga1)
  gbs = (gb0, gb1)
  sems = (sem0, sem1)

  def zero_row(r, carry):
    for j in range(ROW // 16):
      obuf[r, pl.ds(j * 16, 16)] = jnp.zeros((16,), jnp.float32)
    return carry
  lax.fori_loop(0, EB, zero_row, 0)
  for k in range(RPT // EB):
    pltpu.sync_copy(obuf, acc.at[pl.ds(s * RPT + k * EB, EB)])
  rem = RPT - (RPT // EB) * EB
  if rem:
    pltpu.sync_copy(obuf.at[pl.ds(0, rem)],
                    acc.at[pl.ds(s * RPT + (RPT // EB) * EB, rem)])

  plsc.subcore_barrier()

  lanes = lax.iota(jnp.int32, 16)
  keep = lanes < nheads

  def do_chunk(ch, carry):
    pltpu.sync_copy(srcb.at[pl.ds(start_blk + ch * ICH, ICH)], idx_s)
    pltpu.sync_copy(dstb.at[pl.ds(start_blk + ch * ICH, ICH)], idx_d)
    pltpu.async_copy(tabH.at[idx_s.at[0]], gh0, sem0)
    pltpu.async_copy(tabAt.at[idx_s.at[0]], ga0, sem0)
    pltpu.async_copy(tabB.at[idx_d.at[0]], gb0, sem0)

    def do_pair(t, pcarry):
      for q in (0, 1):
        b = t * 2 + q
        nq = 1 - q

        @pl.when(b + 1 < ICH)
        def _prefetch(b=b, nq=nq):
          pltpu.async_copy(tabH.at[idx_s.at[b + 1]], ghs[nq], sems[nq])
          pltpu.async_copy(tabAt.at[idx_s.at[b + 1]], gas[nq], sems[nq])
          pltpu.async_copy(tabB.at[idx_d.at[b + 1]], gbs[nq], sems[nq])

        pltpu.make_async_copy(tabH.at[pl.ds(0, EB)], ghs[q], sems[q]).wait()
        pltpu.make_async_copy(tabAt.at[pl.ds(0, EB)], gas[q], sems[q]).wait()
        pltpu.make_async_copy(tabB.at[pl.ds(0, EB)], gbs[q], sems[q]).wait()

        gh = ghs[q]
        ga = gas[q]
        gb = gbs[q]

        @plsc.parallel_loop(0, EB, step=1, unroll=4)
        def do_edge(e):
          u = ga[e, :]
          v = gb[e, :]
          w = u + v
          a = jnp.where(w > 0, w, 0.2 * w)
          p = jnp.where(keep, jnp.exp(a), 0.0)
          obuf[e, pl.ds(128, 16)] = p
          for jg in range(4):
            hb = gh[e, pl.ds(jg * 32, 32)]
            ha, hc = plsc.unpack(hb, format=plsc.PackFormat.INTERLEAVED)
            pa = _bcast_lane(p, 2 * jg if nheads == HEADS else 0)
            pc = _bcast_lane(p, 2 * jg + 1 if nheads == HEADS else 0)
            obuf[e, pl.ds((2 * jg) * 16, 16)] = pa * ha
            obuf[e, pl.ds((2 * jg + 1) * 16, 16)] = pc * hc

        pltpu.sync_copy(obuf, acc.at[idx_d.at[b]], add=True)
      return pcarry
    lax.fori_loop(0, ICH // 2, do_pair, 0)
    return carry
  lax.fori_loop(0, n_chunks, do_chunk, 0)

  plsc.subcore_barrier()
  pltpu.sync_copy(acc.at[pl.ds(s * RPT, RPT)],
                  out.at[c, pl.ds(s * RPT, RPT)])


def _make_edge_pass(nheads):
  return functools.partial(
      pl.kernel,
      out_type=jax.ShapeDtypeStruct((NC, NPAD, ROW), jnp.float32),
      mesh=_mesh,
      scratch_types=[
          pltpu.VMEM((ICH, EB), jnp.int32),
          pltpu.VMEM((ICH, EB), jnp.int32),
          pltpu.VMEM((EB, 128), jnp.bfloat16),
          pltpu.VMEM((EB, 128), jnp.bfloat16),
          pltpu.VMEM((EB, 16), jnp.float32),
          pltpu.VMEM((EB, 16), jnp.float32),
          pltpu.VMEM((EB, 16), jnp.float32),
          pltpu.VMEM((EB, 16), jnp.float32),
          pltpu.VMEM((EB, ROW), jnp.float32),
          pltpu.VMEM_SHARED((NPAD, ROW), jnp.float32),
          pltpu.SemaphoreType.DMA,
          pltpu.SemaphoreType.DMA,
      ],
      compiler_params=pltpu.CompilerParams(use_tc_tiling_on_sc=False,
                                           needs_layout_passes=False),
  )(functools.partial(_edge_body, nheads))


_edge_pass8 = _make_edge_pass(HEADS)
_edge_pass1 = _make_edge_pass(1)

_BN = 512


def _mm_body(x_ref, wh_ref, wat_ref, wb_ref, oh_ref, oat_ref, ob_ref):
  x = x_ref[...]
  oh_ref[...] = jnp.dot(x, wh_ref[...], preferred_element_type=jnp.float32,
                        precision=lax.Precision.HIGHEST).astype(jnp.bfloat16)
  oat_ref[...] = jnp.dot(x, wat_ref[...], preferred_element_type=jnp.float32,
                         precision=lax.Precision.HIGHEST)
  ob_ref[...] = jnp.dot(x, wb_ref[...], preferred_element_type=jnp.float32,
                        precision=lax.Precision.HIGHEST)


def _tables1(xp, wh, wat, wb):
  return pl.pallas_call(
      _mm_body,
      grid=(NPAD // _BN,),
      in_specs=[
          pl.BlockSpec((_BN, IN_CH), lambda i: (i, 0)),
          pl.BlockSpec((IN_CH, 128), lambda i: (0, 0)),
          pl.BlockSpec((IN_CH, 16), lambda i: (0, 0)),
          pl.BlockSpec((IN_CH, 16), lambda i: (0, 0)),
      ],
      out_specs=[
          pl.BlockSpec((_BN, 128), lambda i: (i, 0)),
          pl.BlockSpec((_BN, 16), lambda i: (i, 0)),
          pl.BlockSpec((_BN, 16), lambda i: (i, 0)),
      ],
      out_shape=[
          jax.ShapeDtypeStruct((NPAD, 128), jnp.bfloat16),
          jax.ShapeDtypeStruct((NPAD, 16), jnp.float32),
          jax.ShapeDtypeStruct((NPAD, 16), jnp.float32),
      ],
  )(xp, wh, wat, wb)


def _prep2_body(a0_ref, a1_ref, r_ref, b1_ref, wh_ref, wat_ref, wb_ref,
                oh_ref, oat_ref, ob_ref):
  acc = a0_ref[...] + a1_ref[...]
  asum = acc[:, 128:136]
  d = jnp.dot(asum, r_ref[...], preferred_element_type=jnp.float32,
              precision=lax.Precision.HIGHEST) + 1e-16
  h1 = acc[:, :128] / d + b1_ref[...]
  e1 = jnp.where(h1 > 0, h1, jnp.exp(h1) - 1.0)
  oh_ref[...] = jnp.dot(e1, wh_ref[...], preferred_element_type=jnp.float32,
                        precision=lax.Precision.HIGHEST).astype(jnp.bfloat16)
  oat_ref[...] = jnp.dot(e1, wat_ref[...], preferred_element_type=jnp.float32,
                         precision=lax.Precision.HIGHEST)
  ob_ref[...] = jnp.dot(e1, wb_ref[...], preferred_element_type=jnp.float32,
                        precision=lax.Precision.HIGHEST)


def _tables2(a0, a1, r, b1, wh, wat, wb):
  return pl.pallas_call(
      _prep2_body,
      grid=(NPAD // _BN,),
      in_specs=[
          pl.BlockSpec((_BN, ROW), lambda i: (i, 0)),
          pl.BlockSpec((_BN, ROW), lambda i: (i, 0)),
          pl.BlockSpec((HEADS, 128), lambda i: (0, 0)),
          pl.BlockSpec((1, 128), lambda i: (0, 0)),
          pl.BlockSpec((IN_CH, 128), lambda i: (0, 0)),
          pl.BlockSpec((IN_CH, 16), lambda i: (0, 0)),
          pl.BlockSpec((IN_CH, 16), lambda i: (0, 0)),
      ],
      out_specs=[
          pl.BlockSpec((_BN, 128), lambda i: (i, 0)),
          pl.BlockSpec((_BN, 16), lambda i: (i, 0)),
          pl.BlockSpec((_BN, 16), lambda i: (i, 0)),
      ],
      out_shape=[
          jax.ShapeDtypeStruct((NPAD, 128), jnp.bfloat16),
          jax.ShapeDtypeStruct((NPAD, 16), jnp.float32),
          jax.ShapeDtypeStruct((NPAD, 16), jnp.float32),
      ],
  )(a0, a1, r, b1, wh, wat, wb)


def _final_body(a0_ref, a1_ref, b2_ref, o_ref):
  acc = a0_ref[...] + a1_ref[...]
  d = acc[:, 128:129] + 1e-16
  o_ref[...] = acc[:, :128] / d + b2_ref[...]


def _final(a0, a1, b2):
  return pl.pallas_call(
      _final_body,
      grid=(NPAD // _BN,),
      in_specs=[
          pl.BlockSpec((_BN, ROW), lambda i: (i, 0)),
          pl.BlockSpec((_BN, ROW), lambda i: (i, 0)),
          pl.BlockSpec((1, 128), lambda i: (0, 0)),
      ],
      out_specs=pl.BlockSpec((_BN, 128), lambda i: (i, 0)),
      out_shape=jax.ShapeDtypeStruct((NPAD, 128), jnp.float32),
  )(a0, a1, b2)


@jax.jit
def kernel(x, W1, att_src1, att_dst1, b1, W2, att_src2, att_dst2, b2,
           edge_index):
  f32 = jnp.float32
  perm = jnp.array(_HPERM, jnp.int32)
  as1 = att_src1.reshape(HEADS, HID)
  ad1 = att_dst1.reshape(HEADS, HID)
  eye8 = jnp.eye(HEADS, dtype=f32)
  A_s = (eye8[:, None, :] * as1[:, :, None]).reshape(IN_CH, HEADS)
  A_d = (eye8[:, None, :] * ad1[:, :, None]).reshape(IN_CH, HEADS)
  w1h = W1[:, perm]
  w1at = jnp.concatenate([W1 @ A_s, W1 @ A_d], axis=1)
  w1b = jnp.concatenate([W1 @ A_d, jnp.zeros((IN_CH, 8), f32)], axis=1)
  a_s2 = att_src2.reshape(OUT_CH)
  a_d2 = att_dst2.reshape(OUT_CH)
  w2h = W2[:, perm]
  w2at = jnp.concatenate(
      [(W2 @ a_s2)[:, None], jnp.zeros((HEADS * HID, 15), f32)], axis=1)
  w2b = jnp.concatenate(
      [(W2 @ a_d2)[:, None], jnp.zeros((HEADS * HID, 15), f32)], axis=1)
  r8 = jnp.repeat(eye8, HID, axis=1)

  loop = jnp.arange(N, dtype=jnp.int32)
  tail = jnp.full((EPAD - N - edge_index.shape[1],), N, jnp.int32)
  src = jnp.concatenate([edge_index[0], loop, tail]).reshape(TOTBLK, EB)
  dst = jnp.concatenate([edge_index[1], loop, tail]).reshape(TOTBLK, EB)

  xp = jnp.pad(x, ((0, NPAD - N), (0, 0)))

  tabH1, tabAt1, tabB1 = _tables1(xp, w1h, w1at, w1b)
  acc1 = _edge_pass8(tabH1, tabAt1, tabB1, src, dst)
  tabH2, tabAt2, tabB2 = _tables2(acc1[0], acc1[1], r8, b1.reshape(1, 128),
                                  w2h, w2at, w2b)
  acc2 = _edge_pass1(tabH2, tabAt2, tabB2, src, dst)
  out = _final(acc2[0], acc2[1], b2.reshape(1, 128))
  return out[:N]


# speedup vs baseline: 1.3101x; 1.0438x over previous
import functools

import jax
import jax.numpy as jnp
from jax import lax
from jax.experimental import pallas as pl
from jax.experimental.pallas import tpu as pltpu
from jax.experimental.pallas import tpu_sc as plsc

N = 10000
IN_CH = 128
HID = 16
HEADS = 8
OUT_CH = 128

NC = 2
NS = 16
NW = NC * NS
EB = 96
ICH = 18
NBLK0 = 126
NBLK1 = 90
TOTBLK = NS * (NBLK0 + NBLK1)
EPAD = TOTBLK * EB
NPAD = 10240
ROW = 144
RPT = NPAD // NS

_HPERM = tuple(
    (2 * jg + (m % 2)) * 16 + (m // 2)
    for jg in range(4) for m in range(32))

_mesh = plsc.VectorSubcoreMesh(core_axis_name="c", subcore_axis_name="s",
                               num_cores=NC, num_subcores=NS)

_GATHER_DN = lax.GatherDimensionNumbers(
    offset_dims=(), collapsed_slice_dims=(0,), start_index_map=(0,))


def _bcast_lane(vec, lane):
  idx = jnp.full((16, 1), lane, jnp.int32)
  return lax.gather(vec, idx, _GATHER_DN, (1,),
                    mode=lax.GatherScatterMode.PROMISE_IN_BOUNDS)


def _edge_body(nheads, tabH, tabAt, tabB, srcb, dstb, out, idx_s, idx_d,
               gh0, gh1, ga0, ga1, gb0, gb1, obuf, acc,
               semh0, semh1, sema0, sema1, semb0, semb1):
  c = lax.axis_index("c")
  s = lax.axis_index("s")
  start_blk = jnp.where(c == 0, s * NBLK0, NS * NBLK0 + s * NBLK1)
  n_chunks = jnp.where(c == 0, NBLK0 // ICH, NBLK1 // ICH)
  ghs = (gh0, gh1)
  gas = (ga0, ga1)
  gbs = (gb0, gb1)
  semh = (semh0, semh1)
  sema = (sema0, sema1)
  semb = (semb0, semb1)

  def zero_row(r, carry):
    for j in range(ROW // 16):
      obuf[r, pl.ds(j * 16, 16)] = jnp.zeros((16,), jnp.float32)
    return carry
  lax.fori_loop(0, EB, zero_row, 0)
  for k in range(RPT // EB):
    pltpu.sync_copy(obuf, acc.at[pl.ds(s * RPT + k * EB, EB)])
  rem = RPT - (RPT // EB) * EB
  if rem:
    pltpu.sync_copy(obuf.at[pl.ds(0, rem)],
                    acc.at[pl.ds(s * RPT + (RPT // EB) * EB, rem)])

  plsc.subcore_barrier()

  lanes = lax.iota(jnp.int32, 16)
  keep = lanes < nheads

  def do_chunk(ch, carry):
    pltpu.sync_copy(srcb.at[pl.ds(start_blk + ch * ICH, ICH)], idx_s)
    pltpu.sync_copy(dstb.at[pl.ds(start_blk + ch * ICH, ICH)], idx_d)
    pltpu.async_copy(tabH.at[idx_s.at[0]], gh0, semh0)
    pltpu.async_copy(tabAt.at[idx_s.at[0]], ga0, sema0)
    pltpu.async_copy(tabB.at[idx_d.at[0]], gb0, semb0)

    def do_pair(t, pcarry):
      for q in (0, 1):
        b = t * 2 + q
        nq = 1 - q

        @pl.when(b + 1 < ICH)
        def _prefetch(b=b, nq=nq):
          pltpu.async_copy(tabH.at[idx_s.at[b + 1]], ghs[nq], semh[nq])
          pltpu.async_copy(tabAt.at[idx_s.at[b + 1]], gas[nq], sema[nq])
          pltpu.async_copy(tabB.at[idx_d.at[b + 1]], gbs[nq], semb[nq])

        pltpu.make_async_copy(tabH.at[pl.ds(0, EB)], ghs[q], semh[q]).wait()
        pltpu.make_async_copy(tabAt.at[pl.ds(0, EB)], gas[q], sema[q]).wait()
        pltpu.make_async_copy(tabB.at[pl.ds(0, EB)], gbs[q], semb[q]).wait()

        gh = ghs[q]
        ga = gas[q]
        gb = gbs[q]

        @plsc.parallel_loop(0, EB, step=1, unroll=4)
        def do_edge(e):
          u = ga[e, :]
          v = gb[e, :]
          w = u + v
          a = jnp.where(w > 0, w, 0.2 * w)
          p = jnp.where(keep, jnp.exp(a), 0.0)
          obuf[e, pl.ds(128, 16)] = p
          p0 = _bcast_lane(p, 0) if nheads != HEADS else None
          for jg in range(4):
            hb = gh[e, pl.ds(jg * 32, 32)]
            ha, hc = plsc.unpack(hb, format=plsc.PackFormat.INTERLEAVED)
            pa = _bcast_lane(p, 2 * jg) if nheads == HEADS else p0
            pc = _bcast_lane(p, 2 * jg + 1) if nheads == HEADS else p0
            obuf[e, pl.ds((2 * jg) * 16, 16)] = pa * ha
            obuf[e, pl.ds((2 * jg + 1) * 16, 16)] = pc * hc

        pltpu.sync_copy(obuf, acc.at[idx_d.at[b]], add=True)
      return pcarry
    lax.fori_loop(0, ICH // 2, do_pair, 0)
    return carry
  lax.fori_loop(0, n_chunks, do_chunk, 0)

  plsc.subcore_barrier()
  pltpu.sync_copy(acc.at[pl.ds(s * RPT, RPT)],
                  out.at[c, pl.ds(s * RPT, RPT)])


def _make_edge_pass(nheads):
  return functools.partial(
      pl.kernel,
      out_type=jax.ShapeDtypeStruct((NC, NPAD, ROW), jnp.float32),
      mesh=_mesh,
      scratch_types=[
          pltpu.VMEM((ICH, EB), jnp.int32),
          pltpu.VMEM((ICH, EB), jnp.int32),
          pltpu.VMEM((EB, 128), jnp.bfloat16),
          pltpu.VMEM((EB, 128), jnp.bfloat16),
          pltpu.VMEM((EB, 16), jnp.float32),
          pltpu.VMEM((EB, 16), jnp.float32),
          pltpu.VMEM((EB, 16), jnp.float32),
          pltpu.VMEM((EB, 16), jnp.float32),
          pltpu.VMEM((EB, ROW), jnp.float32),
          pltpu.VMEM_SHARED((NPAD, ROW), jnp.float32),
          pltpu.SemaphoreType.DMA,
          pltpu.SemaphoreType.DMA,
          pltpu.SemaphoreType.DMA,
          pltpu.SemaphoreType.DMA,
          pltpu.SemaphoreType.DMA,
          pltpu.SemaphoreType.DMA,
      ],
      compiler_params=pltpu.CompilerParams(use_tc_tiling_on_sc=False,
                                           needs_layout_passes=False),
  )(functools.partial(_edge_body, nheads))


_edge_pass8 = _make_edge_pass(HEADS)
_edge_pass1 = _make_edge_pass(1)

_BN = 512


def _mm_body(x_ref, wh_ref, wat_ref, wb_ref, oh_ref, oat_ref, ob_ref):
  x = x_ref[...]
  oh_ref[...] = jnp.dot(x, wh_ref[...], preferred_element_type=jnp.float32,
                        precision=lax.Precision.HIGHEST).astype(jnp.bfloat16)
  oat_ref[...] = jnp.dot(x, wat_ref[...], preferred_element_type=jnp.float32,
                         precision=lax.Precision.HIGHEST)
  ob_ref[...] = jnp.dot(x, wb_ref[...], preferred_element_type=jnp.float32,
                        precision=lax.Precision.HIGHEST)


def _tables1(xp, wh, wat, wb):
  return pl.pallas_call(
      _mm_body,
      grid=(NPAD // _BN,),
      in_specs=[
          pl.BlockSpec((_BN, IN_CH), lambda i: (i, 0)),
          pl.BlockSpec((IN_CH, 128), lambda i: (0, 0)),
          pl.BlockSpec((IN_CH, 16), lambda i: (0, 0)),
          pl.BlockSpec((IN_CH, 16), lambda i: (0, 0)),
      ],
      out_specs=[
          pl.BlockSpec((_BN, 128), lambda i: (i, 0)),
          pl.BlockSpec((_BN, 16), lambda i: (i, 0)),
          pl.BlockSpec((_BN, 16), lambda i: (i, 0)),
      ],
      out_shape=[
          jax.ShapeDtypeStruct((NPAD, 128), jnp.bfloat16),
          jax.ShapeDtypeStruct((NPAD, 16), jnp.float32),
          jax.ShapeDtypeStruct((NPAD, 16), jnp.float32),
      ],
  )(xp, wh, wat, wb)


def _prep2_body(a_ref, r_ref, b1_ref, wh_ref, wat_ref, wb_ref,
                oh_ref, oat_ref, ob_ref):
  acc = a_ref[0] + a_ref[1]
  asum = acc[:, 128:136]
  d = jnp.dot(asum, r_ref[...], preferred_element_type=jnp.float32,
              precision=lax.Precision.HIGHEST) + 1e-16
  h1 = acc[:, :128] / d + b1_ref[...]
  e1 = jnp.where(h1 > 0, h1, jnp.exp(h1) - 1.0)
  oh_ref[...] = jnp.dot(e1, wh_ref[...], preferred_element_type=jnp.float32,
                        precision=lax.Precision.HIGHEST).astype(jnp.bfloat16)
  oat_ref[...] = jnp.dot(e1, wat_ref[...], preferred_element_type=jnp.float32,
                         precision=lax.Precision.HIGHEST)
  ob_ref[...] = jnp.dot(e1, wb_ref[...], preferred_element_type=jnp.float32,
                        precision=lax.Precision.HIGHEST)


def _tables2(accs, r, b1, wh, wat, wb):
  return pl.pallas_call(
      _prep2_body,
      grid=(NPAD // _BN,),
      in_specs=[
          pl.BlockSpec((NC, _BN, ROW), lambda i: (0, i, 0)),
          pl.BlockSpec((HEADS, 128), lambda i: (0, 0)),
          pl.BlockSpec((1, 128), lambda i: (0, 0)),
          pl.BlockSpec((IN_CH, 128), lambda i: (0, 0)),
          pl.BlockSpec((IN_CH, 16), lambda i: (0, 0)),
          pl.BlockSpec((IN_CH, 16), lambda i: (0, 0)),
      ],
      out_specs=[
          pl.BlockSpec((_BN, 128), lambda i: (i, 0)),
          pl.BlockSpec((_BN, 16), lambda i: (i, 0)),
          pl.BlockSpec((_BN, 16), lambda i: (i, 0)),
      ],
      out_shape=[
          jax.ShapeDtypeStruct((NPAD, 128), jnp.bfloat16),
          jax.ShapeDtypeStruct((NPAD, 16), jnp.float32),
          jax.ShapeDtypeStruct((NPAD, 16), jnp.float32),
      ],
  )(accs, r, b1, wh, wat, wb)


def _final_body(a_ref, b2_ref, o_ref):
  acc = a_ref[0] + a_ref[1]
  d = acc[:, 128:129] + 1e-16
  o_ref[...] = acc[:, :128] / d + b2_ref[...]


def _final(accs, b2):
  return pl.pallas_call(
      _final_body,
      grid=(NPAD // _BN,),
      in_specs=[
          pl.BlockSpec((NC, _BN, ROW), lambda i: (0, i, 0)),
          pl.BlockSpec((1, 128), lambda i: (0, 0)),
      ],
      out_specs=pl.BlockSpec((_BN, 128), lambda i: (i, 0)),
      out_shape=jax.ShapeDtypeStruct((NPAD, 128), jnp.float32),
  )(accs, b2)


@jax.jit
def kernel(x, W1, att_src1, att_dst1, b1, W2, att_src2, att_dst2, b2,
           edge_index):
  f32 = jnp.float32
  perm = jnp.array(_HPERM, jnp.int32)
  as1 = att_src1.reshape(HEADS, HID)
  ad1 = att_dst1.reshape(HEADS, HID)
  eye8 = jnp.eye(HEADS, dtype=f32)
  A_s = (eye8[:, None, :] * as1[:, :, None]).reshape(IN_CH, HEADS)
  A_d = (eye8[:, None, :] * ad1[:, :, None]).reshape(IN_CH, HEADS)
  w1h = W1[:, perm]
  w1at = jnp.concatenate([W1 @ A_s, W1 @ A_d], axis=1)
  w1b = jnp.concatenate([W1 @ A_d, jnp.zeros((IN_CH, 8), f32)], axis=1)
  a_s2 = att_src2.reshape(OUT_CH)
  a_d2 = att_dst2.reshape(OUT_CH)
  w2h = W2[:, perm]
  w2at = jnp.concatenate(
      [(W2 @ a_s2)[:, None], jnp.zeros((HEADS * HID, 15), f32)], axis=1)
  w2b = jnp.concatenate(
      [(W2 @ a_d2)[:, None], jnp.zeros((HEADS * HID, 15), f32)], axis=1)
  r8 = jnp.repeat(eye8, HID, axis=1)

  loop = jnp.arange(N, dtype=jnp.int32)
  tail = jnp.full((EPAD - N - edge_index.shape[1],), N, jnp.int32)
  src = jnp.concatenate([edge_index[0], loop, tail]).reshape(TOTBLK, EB)
  dst = jnp.concatenate([edge_index[1], loop, tail]).reshape(TOTBLK, EB)

  xp = jnp.pad(x, ((0, NPAD - N), (0, 0)))

  tabH1, tabAt1, tabB1 = _tables1(xp, w1h, w1at, w1b)
  acc1 = _edge_pass8(tabH1, tabAt1, tabB1, src, dst)
  tabH2, tabAt2, tabB2 = _tables2(acc1, r8, b1.reshape(1, 128),
                                  w2h, w2at, w2b)
  acc2 = _edge_pass1(tabH2, tabAt2, tabB2, src, dst)
  out = _final(acc2, b2.reshape(1, 128))
  return out[:N]
